# baseline (speedup 1.0000x reference)
import functools

import jax
import jax.numpy as jnp
from jax import lax
from jax.experimental import pallas as pl
from jax.experimental.pallas import tpu as pltpu
from jax.experimental.pallas import tpu_sc as plsc

TOTAL = 32768
D = 1024
NSEG = 16
NC = 2
NS = 16
NW = NC * NS
T = TOTAL // NW
R = 32
NTILES = T // R
G = 16
NCH = D // 16
NEG = -1e30


def _sc_partials(x, w, cu_pad):
    mesh = plsc.VectorSubcoreMesh(core_axis_name="c", subcore_axis_name="s",
                                  num_cores=NC, num_subcores=NS)

    @functools.partial(
        pl.kernel,
        out_type=(
            jax.ShapeDtypeStruct((NW, NSEG * D), jnp.float32),
            jax.ShapeDtypeStruct((NW, NSEG * 16), jnp.float32),
            jax.ShapeDtypeStruct((NW, NSEG * 16), jnp.float32),
        ),
        mesh=mesh,
        compiler_params=pltpu.CompilerParams(needs_layout_passes=False,
                                             use_tc_tiling_on_sc=True),
        scratch_types=[
            pltpu.VMEM((R, D), jnp.float32),
            pltpu.VMEM((R, D), jnp.float32),
            pltpu.VMEM((D,), jnp.float32),
            pltpu.VMEM((48,), jnp.int32),
            pltpu.VMEM((NSEG * D,), jnp.float32),
            pltpu.VMEM((D,), jnp.float32),
            pltpu.VMEM((16,), jnp.float32),
            pltpu.VMEM((NSEG * 16,), jnp.float32),
            pltpu.VMEM((NSEG * 16,), jnp.float32),
            pltpu.SMEM((1,), jnp.float32),
            pltpu.SMEM((1,), jnp.int32),
            pltpu.SMEM((1,), jnp.int32),
            pltpu.SemaphoreType.DMA,
            pltpu.SemaphoreType.DMA,
        ],
    )
    def k(x_hbm, w_hbm, cu_hbm, acc_out, m_out, s_out,
          xbuf0, xbuf1, wv, cuv, acc, cacc, sv, mloc, sloc,
          mref, cur_seg, nxt, sem0, sem1):
        wid = lax.axis_index("s") * NC + lax.axis_index("c")
        t0 = wid * T

        pltpu.sync_copy(w_hbm, wv)
        pltpu.sync_copy(cu_hbm, cuv)

        zeros = jnp.zeros((16,), jnp.float32)
        iota16 = lax.iota(jnp.int32, 16)

        def zacc(i, _):
            acc[pl.ds(i * 16, 16)] = zeros
            return _
        lax.fori_loop(0, NSEG * NCH, zacc, None)

        def zcacc(i, _):
            cacc[pl.ds(i * 16, 16)] = zeros
            return _
        lax.fori_loop(0, NCH, zcacc, None)

        negs = jnp.full((16,), NEG, jnp.float32)

        def zloc(i, _):
            mloc[pl.ds(i * 16, 16)] = negs
            sloc[pl.ds(i * 16, 16)] = zeros
            return _
        lax.fori_loop(0, NSEG, zloc, None)

        sv[...] = zeros
        mref[0] = jnp.float32(NEG)

        cu_lo = cuv[pl.ds(0, 16)]
        seg0 = jnp.int32(0)
        for kk in range(1, NSEG):
            seg0 = seg0 + jnp.where(t0 >= cu_lo[kk], 1, 0).astype(jnp.int32)
        cur_seg[0] = seg0
        nxt[0] = cuv[pl.ds(seg0 + 1, 16)][0]

        def flush():
            cs = cur_seg[0]
            mloc[pl.ds(cs * 16, 16)] = jnp.full((16,), mref[0], jnp.float32)
            sloc[pl.ds(cs * 16, 16)] = sv[...]

            def cp(j, _):
                acc[pl.ds(cs * D + j * 16, 16)] = cacc[pl.ds(j * 16, 16)]
                cacc[pl.ds(j * 16, 16)] = zeros
                return _
            lax.fori_loop(0, NCH, cp, None)
            sv[...] = zeros
            mref[0] = jnp.float32(NEG)
            cur_seg[0] = cs + 1
            nxt[0] = cuv[pl.ds(cs + 2, 16)][0]

        def process(xbuf, base_row):
            for grp in range(R // G):
                gr = grp * G
                g0 = base_row + gr

                @plsc.parallel_loop(0, NCH, unroll=2,
                                    carry=tuple(zeros for _ in range(G)))
                def accs(j, carry):
                    wch = wv[pl.ds(j * 16, 16)]
                    return tuple(
                        carry[r] + xbuf[gr + r, pl.ds(j * 16, 16)] * wch
                        for r in range(G))
                S = zeros
                for r in range(G):
                    S = jnp.where(iota16 == r, jnp.sum(accs[r]), S)

                no_boundary = nxt[0] >= g0 + G

                @pl.when(no_boundary)
                def _():
                    m_old = mref[0]
                    m_g = jnp.max(S)
                    m_new = jnp.maximum(m_old, m_g)
                    f_v = jnp.exp(jnp.full((16,), m_old - m_new, jnp.float32))
                    C = jnp.exp(S - m_new)
                    sv[...] = sv[...] * f_v + jnp.full((16,), jnp.sum(C),
                                                       jnp.float32)
                    mref[0] = m_new
                    cb = [jnp.full((16,), C[r], jnp.float32)
                          for r in range(G)]

                    @plsc.parallel_loop(0, NCH, unroll=2)
                    def _upd(j):
                        p = [cb[r] * xbuf[gr + r, pl.ds(j * 16, 16)]
                             for r in range(4)]
                        for r in range(4, G):
                            p[r % 4] = (p[r % 4]
                                        + cb[r] * xbuf[gr + r,
                                                       pl.ds(j * 16, 16)])
                        old = cacc[pl.ds(j * 16, 16)] * f_v
                        cacc[pl.ds(j * 16, 16)] = (
                            ((p[0] + p[1]) + (p[2] + p[3])) + old)

                @pl.when(jnp.logical_not(no_boundary))
                def _():
                    def row(r, _):
                        g = g0 + r

                        @pl.when(g == nxt[0])
                        def _():
                            flush()

                        sc = jnp.max(jnp.where(iota16 == r, S, negs))
                        m_old = mref[0]
                        m_new = jnp.maximum(m_old, sc)
                        f_v = jnp.exp(jnp.full((16,), m_old - m_new,
                                               jnp.float32))
                        c_v = jnp.exp(jnp.full((16,), sc - m_new, jnp.float32))
                        sv[...] = sv[...] * f_v + c_v
                        mref[0] = m_new

                        def upd(j, _):
                            cacc[pl.ds(j * 16, 16)] = (
                                cacc[pl.ds(j * 16, 16)] * f_v
                                + c_v * xbuf[gr + r, pl.ds(j * 16, 16)])
                            return _
                        lax.fori_loop(0, NCH, upd, None)
                        return _
                    lax.fori_loop(0, G, row, None)

        pltpu.async_copy(x_hbm.at[pl.ds(t0, R), :], xbuf0, sem0)

        def tile_pair(kk, _):
            r1 = t0 + (2 * kk + 1) * R
            pltpu.async_copy(x_hbm.at[pl.ds(r1, R), :], xbuf1, sem1)
            pltpu.make_async_copy(x_hbm.at[pl.ds(0, R), :], xbuf0, sem0).wait()
            process(xbuf0, t0 + (2 * kk) * R)
            nx = t0 + jnp.minimum((2 * kk + 2) * R, (NTILES - 1) * R)
            pltpu.async_copy(x_hbm.at[pl.ds(nx, R), :], xbuf0, sem0)
            pltpu.make_async_copy(x_hbm.at[pl.ds(0, R), :], xbuf1, sem1).wait()
            process(xbuf1, t0 + (2 * kk + 1) * R)
            return _

        lax.fori_loop(0, NTILES // 2, tile_pair, None)
        pltpu.make_async_copy(x_hbm.at[pl.ds(0, R), :], xbuf0, sem0).wait()

        flush()

        pltpu.sync_copy(acc, acc_out.at[wid])
        pltpu.sync_copy(mloc, m_out.at[wid])
        pltpu.sync_copy(sloc, s_out.at[wid])

    return k(x, w, cu_pad)


def _combine_kernel(acc_ref, m_ref, s_ref, out_ref):
    m = m_ref[:, :, 0]
    s = s_ref[:, :, 0]
    gmax = jnp.max(m, axis=0)
    f = jnp.exp(m - gmax[None, :])
    denom = jnp.sum(f * s, axis=0)
    wacc = jnp.sum(acc_ref[...] * f[:, :, None], axis=0)
    out_ref[...] = wacc / denom[:, None]


def kernel(x, cu_seqlens, W, b):
    w = W.reshape(-1)
    cu_pad = jnp.concatenate(
        [cu_seqlens.astype(jnp.int32),
         jnp.full((48 - NSEG - 1,), TOTAL, jnp.int32)])
    acc_p, m_p, s_p = _sc_partials(x, w, cu_pad)
    out = pl.pallas_call(
        _combine_kernel,
        out_shape=jax.ShapeDtypeStruct((NSEG, D), jnp.float32),
    )(acc_p.reshape(NW, NSEG, D), m_p.reshape(NW, NSEG, 16),
      s_p.reshape(NW, NSEG, 16))
    return out


# speedup vs baseline: 1.0458x; 1.0458x over previous
import functools

import jax
import jax.numpy as jnp
from jax import lax
from jax.experimental import pallas as pl
from jax.experimental.pallas import tpu as pltpu
from jax.experimental.pallas import tpu_sc as plsc

TOTAL = 32768
D = 1024
NSEG = 16
NC = 2
NS = 16
NW = NC * NS
T = TOTAL // NW
R = 32
NTILES = T // R
G = 16
NCH = D // 16
NEG = -1e30


def _sc_partials(x, w, cu):
    mesh = plsc.VectorSubcoreMesh(core_axis_name="c", subcore_axis_name="s",
                                  num_cores=NC, num_subcores=NS)

    @functools.partial(
        pl.kernel,
        out_type=(
            jax.ShapeDtypeStruct((NW, NSEG, D), jnp.float32),
            jax.ShapeDtypeStruct((NW, NSEG, 16), jnp.float32),
            jax.ShapeDtypeStruct((NW, NSEG, 16), jnp.float32),
        ),
        mesh=mesh,
        compiler_params=pltpu.CompilerParams(needs_layout_passes=False,
                                             use_tc_tiling_on_sc=True),
        scratch_types=[
            pltpu.VMEM((R, D), jnp.float32),
            pltpu.VMEM((R, D), jnp.float32),
            pltpu.VMEM((D,), jnp.float32),
            pltpu.VMEM((NSEG + 1,), jnp.int32),
            pltpu.VMEM((NSEG, D), jnp.float32),
            pltpu.VMEM((D,), jnp.float32),
            pltpu.VMEM((16,), jnp.float32),
            pltpu.VMEM((NSEG, 16), jnp.float32),
            pltpu.VMEM((NSEG, 16), jnp.float32),
            pltpu.SMEM((1,), jnp.float32),
            pltpu.SMEM((1,), jnp.int32),
            pltpu.SMEM((1,), jnp.int32),
            pltpu.SemaphoreType.DMA,
            pltpu.SemaphoreType.DMA,
        ],
    )
    def k(x_hbm, w_hbm, cu_hbm, acc_out, m_out, s_out,
          xbuf0, xbuf1, wv, cuv, acc, cacc, sv, mloc, sloc,
          mref, cur_seg, nxt, sem0, sem1):
        wid = lax.axis_index("s") * NC + lax.axis_index("c")
        t0 = wid * T

        pltpu.sync_copy(w_hbm, wv)
        pltpu.sync_copy(cu_hbm, cuv)

        zeros = jnp.zeros((16,), jnp.float32)
        iota16 = lax.iota(jnp.int32, 16)

        def zacc(i, _):
            def zrow(j, _):
                acc[i, pl.ds(j * 16, 16)] = zeros
                return _
            lax.fori_loop(0, NCH, zrow, None)
            return _
        lax.fori_loop(0, NSEG, zacc, None)

        def zcacc(i, _):
            cacc[pl.ds(i * 16, 16)] = zeros
            return _
        lax.fori_loop(0, NCH, zcacc, None)

        negs = jnp.full((16,), NEG, jnp.float32)

        def zloc(i, _):
            mloc[i, :] = negs
            sloc[i, :] = zeros
            return _
        lax.fori_loop(0, NSEG, zloc, None)

        sv[...] = zeros
        mref[0] = jnp.float32(NEG)

        cu_hi = cuv[pl.ds(1, 16)]

        seg0 = jnp.int32(0)
        for kk in range(NSEG - 1):
            seg0 = seg0 + jnp.where(t0 >= cu_hi[kk], 1, 0).astype(jnp.int32)
        cur_seg[0] = seg0
        nxt[0] = jnp.max(jnp.where(iota16 == seg0, cu_hi, 0))

        def flush():
            cs = cur_seg[0]
            mloc[cs, :] = jnp.full((16,), mref[0], jnp.float32)
            sloc[cs, :] = sv[...]

            def cp(j, _):
                acc[cs, pl.ds(j * 16, 16)] = cacc[pl.ds(j * 16, 16)]
                cacc[pl.ds(j * 16, 16)] = zeros
                return _
            lax.fori_loop(0, NCH, cp, None)
            sv[...] = zeros
            mref[0] = jnp.float32(NEG)
            cur_seg[0] = cs + 1
            nxt[0] = jnp.max(jnp.where(iota16 == cs + 1, cu_hi, 0))

        def process(xbuf, base_row):
            for grp in range(R // G):
                gr = grp * G
                g0 = base_row + gr

                @plsc.parallel_loop(0, NCH, carry=tuple(zeros
                                                        for _ in range(G)))
                def accs(j, carry):
                    wch = wv[pl.ds(j * 16, 16)]
                    return tuple(
                        carry[r] + xbuf[gr + r, pl.ds(j * 16, 16)] * wch
                        for r in range(G))
                S = zeros
                for r in range(G):
                    S = jnp.where(iota16 == r, jnp.sum(accs[r]), S)

                no_boundary = nxt[0] >= g0 + G

                @pl.when(no_boundary)
                def _():
                    m_old = mref[0]
                    m_g = jnp.max(S)
                    m_new = jnp.maximum(m_old, m_g)
                    f_v = jnp.exp(jnp.full((16,), m_old - m_new, jnp.float32))
                    C = jnp.exp(S - m_new)
                    sv[...] = sv[...] * f_v + jnp.full((16,), jnp.sum(C),
                                                       jnp.float32)
                    mref[0] = m_new
                    cb = [jnp.full((16,), C[r], jnp.float32)
                          for r in range(G)]

                    @plsc.parallel_loop(0, NCH)
                    def _upd(j):
                        p = [cb[r] * xbuf[gr + r, pl.ds(j * 16, 16)]
                             for r in range(4)]
                        for r in range(4, G):
                            p[r % 4] = (p[r % 4]
                                        + cb[r] * xbuf[gr + r,
                                                       pl.ds(j * 16, 16)])
                        old = cacc[pl.ds(j * 16, 16)] * f_v
                        cacc[pl.ds(j * 16, 16)] = (
                            ((p[0] + p[1]) + (p[2] + p[3])) + old)

                @pl.when(jnp.logical_not(no_boundary))
                def _():
                    def row(r, _):
                        g = g0 + r

                        @pl.when(g == nxt[0])
                        def _():
                            flush()

                        sc = jnp.max(jnp.where(iota16 == r, S, negs))
                        m_old = mref[0]
                        m_new = jnp.maximum(m_old, sc)
                        f_v = jnp.exp(jnp.full((16,), m_old - m_new,
                                               jnp.float32))
                        c_v = jnp.exp(jnp.full((16,), sc - m_new, jnp.float32))
                        sv[...] = sv[...] * f_v + c_v
                        mref[0] = m_new

                        def upd(j, _):
                            cacc[pl.ds(j * 16, 16)] = (
                                cacc[pl.ds(j * 16, 16)] * f_v
                                + c_v * xbuf[gr + r, pl.ds(j * 16, 16)])
                            return _
                        lax.fori_loop(0, NCH, upd, None)
                        return _
                    lax.fori_loop(0, G, row, None)

        pltpu.async_copy(x_hbm.at[pl.ds(t0, R), :], xbuf0, sem0)

        def tile_pair(kk, _):
            r1 = t0 + (2 * kk + 1) * R
            pltpu.async_copy(x_hbm.at[pl.ds(r1, R), :], xbuf1, sem1)
            pltpu.make_async_copy(x_hbm.at[pl.ds(0, R), :], xbuf0, sem0).wait()
            process(xbuf0, t0 + (2 * kk) * R)
            nx = t0 + jnp.minimum((2 * kk + 2) * R, (NTILES - 1) * R)
            pltpu.async_copy(x_hbm.at[pl.ds(nx, R), :], xbuf0, sem0)
            pltpu.make_async_copy(x_hbm.at[pl.ds(0, R), :], xbuf1, sem1).wait()
            process(xbuf1, t0 + (2 * kk + 1) * R)
            return _

        lax.fori_loop(0, NTILES // 2, tile_pair, None)
        pltpu.make_async_copy(x_hbm.at[pl.ds(0, R), :], xbuf0, sem0).wait()

        flush()

        pltpu.sync_copy(acc, acc_out.at[wid])
        pltpu.sync_copy(mloc, m_out.at[wid])
        pltpu.sync_copy(sloc, s_out.at[wid])

    return k(x, w, cu)


def _combine_kernel(acc_ref, m_ref, s_ref, out_ref):
    m = m_ref[:, :, 0]
    s = s_ref[:, :, 0]
    gmax = jnp.max(m, axis=0)
    f = jnp.exp(m - gmax[None, :])
    denom = jnp.sum(f * s, axis=0)
    wacc = jnp.sum(acc_ref[...] * f[:, :, None], axis=0)
    out_ref[...] = wacc / denom[:, None]


def kernel(x, cu_seqlens, W, b):
    w = W.reshape(-1)
    acc_p, m_p, s_p = _sc_partials(x, w, cu_seqlens.astype(jnp.int32))
    out = pl.pallas_call(
        _combine_kernel,
        out_shape=jax.ShapeDtypeStruct((NSEG, D), jnp.float32),
    )(acc_p, m_p, s_p)
    return out


# speedup vs baseline: 1.0509x; 1.0049x over previous
import functools

import jax
import jax.numpy as jnp
from jax import lax
from jax.experimental import pallas as pl
from jax.experimental.pallas import tpu as pltpu
from jax.experimental.pallas import tpu_sc as plsc

TOTAL = 32768
D = 1024
NSEG = 16
NC = 2
NS = 16
NW = NC * NS
T = TOTAL // NW
R = 32
NTILES = T // R
G = 16
NCH = D // 16
NEG = -1e30


def _sc_partials(x, w, cu):
    mesh = plsc.VectorSubcoreMesh(core_axis_name="c", subcore_axis_name="s",
                                  num_cores=NC, num_subcores=NS)

    @functools.partial(
        pl.kernel,
        out_type=(
            jax.ShapeDtypeStruct((NW, NSEG, D), jnp.float32),
            jax.ShapeDtypeStruct((NW, NSEG), jnp.float32),
            jax.ShapeDtypeStruct((NW, NSEG), jnp.float32),
        ),
        mesh=mesh,
        compiler_params=pltpu.CompilerParams(needs_layout_passes=False,
                                             use_tc_tiling_on_sc=True),
        scratch_types=[
            pltpu.VMEM((R, D), jnp.float32),
            pltpu.VMEM((R, D), jnp.float32),
            pltpu.VMEM((D,), jnp.float32),
            pltpu.VMEM((NSEG + 1,), jnp.int32),
            pltpu.VMEM((NSEG, D), jnp.float32),
            pltpu.VMEM((D,), jnp.float32),
            pltpu.VMEM((16,), jnp.float32),
            pltpu.VMEM((NSEG,), jnp.float32),
            pltpu.VMEM((NSEG,), jnp.float32),
            pltpu.SMEM((1,), jnp.float32),
            pltpu.SMEM((1,), jnp.int32),
            pltpu.SMEM((1,), jnp.int32),
            pltpu.SemaphoreType.DMA,
            pltpu.SemaphoreType.DMA,
        ],
    )
    def k(x_hbm, w_hbm, cu_hbm, acc_out, m_out, s_out,
          xbuf0, xbuf1, wv, cuv, acc, cacc, sv, mloc, sloc,
          mref, cur_seg, nxt, sem0, sem1):
        wid = lax.axis_index("s") * NC + lax.axis_index("c")
        t0 = wid * T

        pltpu.sync_copy(w_hbm, wv)
        pltpu.sync_copy(cu_hbm, cuv)

        zeros = jnp.zeros((16,), jnp.float32)
        iota16 = lax.iota(jnp.int32, 16)

        def zacc(i, _):
            def zrow(j, _):
                acc[i, pl.ds(j * 16, 16)] = zeros
                return _
            lax.fori_loop(0, NCH, zrow, None)
            return _
        lax.fori_loop(0, NSEG, zacc, None)

        def zcacc(i, _):
            cacc[pl.ds(i * 16, 16)] = zeros
            return _
        lax.fori_loop(0, NCH, zcacc, None)

        negs = jnp.full((16,), NEG, jnp.float32)

        mloc[...] = negs
        sloc[...] = zeros

        sv[...] = zeros
        mref[0] = jnp.float32(NEG)

        cu_hi = cuv[pl.ds(1, 16)]

        seg0 = jnp.int32(0)
        for kk in range(NSEG - 1):
            seg0 = seg0 + jnp.where(t0 >= cu_hi[kk], 1, 0).astype(jnp.int32)
        cur_seg[0] = seg0
        nxt[0] = jnp.max(jnp.where(iota16 == seg0, cu_hi, 0))

        def flush():
            cs = cur_seg[0]
            seg_lane = iota16 == cs
            mloc[...] = jnp.where(seg_lane,
                                  jnp.full((16,), mref[0], jnp.float32),
                                  mloc[...])
            sloc[...] = jnp.where(seg_lane, sv[...], sloc[...])

            def cp(j, _):
                acc[cs, pl.ds(j * 16, 16)] = cacc[pl.ds(j * 16, 16)]
                cacc[pl.ds(j * 16, 16)] = zeros
                return _
            lax.fori_loop(0, NCH, cp, None)
            sv[...] = zeros
            mref[0] = jnp.float32(NEG)
            cur_seg[0] = cs + 1
            nxt[0] = jnp.max(jnp.where(iota16 == cs + 1, cu_hi, 0))

        def process(xbuf, base_row):
            for grp in range(R // G):
                gr = grp * G
                g0 = base_row + gr

                @plsc.parallel_loop(0, NCH, carry=tuple(zeros
                                                        for _ in range(G)))
                def accs(j, carry):
                    wch = wv[pl.ds(j * 16, 16)]
                    return tuple(
                        carry[r] + xbuf[gr + r, pl.ds(j * 16, 16)] * wch
                        for r in range(G))
                S = zeros
                for r in range(G):
                    S = jnp.where(iota16 == r, jnp.sum(accs[r]), S)

                no_boundary = nxt[0] >= g0 + G

                @pl.when(no_boundary)
                def _():
                    m_old = mref[0]
                    m_g = jnp.max(S)
                    m_new = jnp.maximum(m_old, m_g)
                    f_v = jnp.exp(jnp.full((16,), m_old - m_new, jnp.float32))
                    C = jnp.exp(S - m_new)
                    sv[...] = sv[...] * f_v + jnp.full((16,), jnp.sum(C),
                                                       jnp.float32)
                    mref[0] = m_new
                    cb = [jnp.full((16,), C[r], jnp.float32)
                          for r in range(G)]

                    @plsc.parallel_loop(0, NCH)
                    def _upd(j):
                        p = [cb[r] * xbuf[gr + r, pl.ds(j * 16, 16)]
                             for r in range(4)]
                        for r in range(4, G):
                            p[r % 4] = (p[r % 4]
                                        + cb[r] * xbuf[gr + r,
                                                       pl.ds(j * 16, 16)])
                        old = cacc[pl.ds(j * 16, 16)] * f_v
                        cacc[pl.ds(j * 16, 16)] = (
                            ((p[0] + p[1]) + (p[2] + p[3])) + old)

                @pl.when(jnp.logical_not(no_boundary))
                def _():
                    def row(r, _):
                        g = g0 + r

                        @pl.when(g == nxt[0])
                        def _():
                            flush()

                        sc = jnp.max(jnp.where(iota16 == r, S, negs))
                        m_old = mref[0]
                        m_new = jnp.maximum(m_old, sc)
                        f_v = jnp.exp(jnp.full((16,), m_old - m_new,
                                               jnp.float32))
                        c_v = jnp.exp(jnp.full((16,), sc - m_new, jnp.float32))
                        sv[...] = sv[...] * f_v + c_v
                        mref[0] = m_new

                        def upd(j, _):
                            cacc[pl.ds(j * 16, 16)] = (
                                cacc[pl.ds(j * 16, 16)] * f_v
                                + c_v * xbuf[gr + r, pl.ds(j * 16, 16)])
                            return _
                        lax.fori_loop(0, NCH, upd, None)
                        return _
                    lax.fori_loop(0, G, row, None)

        pltpu.async_copy(x_hbm.at[pl.ds(t0, R), :], xbuf0, sem0)

        def tile_pair(kk, _):
            r1 = t0 + (2 * kk + 1) * R
            pltpu.async_copy(x_hbm.at[pl.ds(r1, R), :], xbuf1, sem1)
            pltpu.make_async_copy(x_hbm.at[pl.ds(0, R), :], xbuf0, sem0).wait()
            process(xbuf0, t0 + (2 * kk) * R)
            nx = t0 + jnp.minimum((2 * kk + 2) * R, (NTILES - 1) * R)
            pltpu.async_copy(x_hbm.at[pl.ds(nx, R), :], xbuf0, sem0)
            pltpu.make_async_copy(x_hbm.at[pl.ds(0, R), :], xbuf1, sem1).wait()
            process(xbuf1, t0 + (2 * kk + 1) * R)
            return _

        lax.fori_loop(0, NTILES // 2, tile_pair, None)
        pltpu.make_async_copy(x_hbm.at[pl.ds(0, R), :], xbuf0, sem0).wait()

        flush()

        pltpu.sync_copy(acc, acc_out.at[wid])
        pltpu.sync_copy(mloc, m_out.at[wid])
        pltpu.sync_copy(sloc, s_out.at[wid])

    return k(x, w, cu)


def _combine_kernel(acc_ref, m_ref, s_ref, out_ref):
    m = m_ref[...]
    s = s_ref[...]
    gmax = jnp.max(m, axis=0)
    f = jnp.exp(m - gmax[None, :])
    denom = jnp.sum(f * s, axis=0)
    wacc = jnp.sum(acc_ref[...] * f[:, :, None], axis=0)
    out_ref[...] = wacc / denom[:, None]


def kernel(x, cu_seqlens, W, b):
    w = W.reshape(-1)
    acc_p, m_p, s_p = _sc_partials(x, w, cu_seqlens.astype(jnp.int32))
    out = pl.pallas_call(
        _combine_kernel,
        out_shape=jax.ShapeDtypeStruct((NSEG, D), jnp.float32),
    )(acc_p, m_p, s_p)
    return out


# speedup vs baseline: 1.0816x; 1.0292x over previous
import functools

import jax
import jax.numpy as jnp
from jax import lax
from jax.experimental import pallas as pl
from jax.experimental.pallas import tpu as pltpu
from jax.experimental.pallas import tpu_sc as plsc

TOTAL = 32768
D = 1024
NSEG = 16
NC = 2
NS = 16
NW = NC * NS
T = TOTAL // NW
R = 32
NTILES = T // R
G = 16
NCH = D // 16
NEG = -1e30


def _sc_partials(x, w, cu):
    mesh = plsc.VectorSubcoreMesh(core_axis_name="c", subcore_axis_name="s",
                                  num_cores=NC, num_subcores=NS)

    @functools.partial(
        pl.kernel,
        out_type=(
            jax.ShapeDtypeStruct((NW, NSEG, D), jnp.float32),
            jax.ShapeDtypeStruct((NW, NSEG), jnp.float32),
            jax.ShapeDtypeStruct((NW, NSEG), jnp.float32),
        ),
        mesh=mesh,
        compiler_params=pltpu.CompilerParams(needs_layout_passes=False,
                                             use_tc_tiling_on_sc=True),
        scratch_types=[
            pltpu.VMEM((R, D), jnp.float32),
            pltpu.VMEM((R, D), jnp.float32),
            pltpu.VMEM((D,), jnp.float32),
            pltpu.VMEM((NSEG + 1,), jnp.int32),
            pltpu.VMEM((NSEG, D), jnp.float32),
            pltpu.VMEM((D,), jnp.float32),
            pltpu.VMEM((16,), jnp.float32),
            pltpu.VMEM((NSEG,), jnp.float32),
            pltpu.VMEM((NSEG,), jnp.float32),
            pltpu.SMEM((1,), jnp.float32),
            pltpu.SMEM((1,), jnp.int32),
            pltpu.SMEM((1,), jnp.int32),
            pltpu.SemaphoreType.DMA,
            pltpu.SemaphoreType.DMA,
        ],
    )
    def k(x_hbm, w_hbm, cu_hbm, acc_out, m_out, s_out,
          xbuf0, xbuf1, wv, cuv, acc, cacc, sv, mloc, sloc,
          mref, cur_seg, nxt, sem0, sem1):
        wid = lax.axis_index("s") * NC + lax.axis_index("c")
        t0 = wid * T

        pltpu.sync_copy(w_hbm, wv)
        pltpu.sync_copy(cu_hbm, cuv)

        zeros = jnp.zeros((16,), jnp.float32)
        iota16 = lax.iota(jnp.int32, 16)

        def zcacc(i, _):
            cacc[pl.ds(i * 16, 16)] = zeros
            return _
        lax.fori_loop(0, NCH, zcacc, None)

        negs = jnp.full((16,), NEG, jnp.float32)

        mloc[...] = negs
        sloc[...] = zeros

        sv[...] = zeros
        mref[0] = jnp.float32(NEG)

        cu_hi = cuv[pl.ds(1, 16)]

        seg0 = jnp.int32(0)
        for kk in range(NSEG - 1):
            seg0 = seg0 + jnp.where(t0 >= cu_hi[kk], 1, 0).astype(jnp.int32)
        cur_seg[0] = seg0
        nxt[0] = jnp.max(jnp.where(iota16 == seg0, cu_hi, 0))

        def flush():
            cs = cur_seg[0]
            seg_lane = iota16 == cs
            mloc[...] = jnp.where(seg_lane,
                                  jnp.full((16,), mref[0], jnp.float32),
                                  mloc[...])
            sloc[...] = jnp.where(seg_lane, sv[...], sloc[...])

            def cp(j, _):
                acc[cs, pl.ds(j * 16, 16)] = cacc[pl.ds(j * 16, 16)]
                cacc[pl.ds(j * 16, 16)] = zeros
                return _
            lax.fori_loop(0, NCH, cp, None)
            sv[...] = zeros
            mref[0] = jnp.float32(NEG)
            cur_seg[0] = cs + 1
            nxt[0] = jnp.max(jnp.where(iota16 == cs + 1, cu_hi, 0))

        def process(xbuf, base_row):
            for grp in range(R // G):
                gr = grp * G
                g0 = base_row + gr

                @plsc.parallel_loop(0, NCH, carry=tuple(zeros
                                                        for _ in range(G)))
                def accs(j, carry):
                    wch = wv[pl.ds(j * 16, 16)]
                    return tuple(
                        carry[r] + xbuf[gr + r, pl.ds(j * 16, 16)] * wch
                        for r in range(G))
                S = zeros
                for r in range(G):
                    S = jnp.where(iota16 == r, jnp.sum(accs[r]), S)

                no_boundary = nxt[0] >= g0 + G

                @pl.when(no_boundary)
                def _():
                    m_old = mref[0]
                    m_g = jnp.max(S)
                    m_new = jnp.maximum(m_old, m_g)
                    f_v = jnp.exp(jnp.full((16,), m_old - m_new, jnp.float32))
                    C = jnp.exp(S - m_new)
                    sv[...] = sv[...] * f_v + jnp.full((16,), jnp.sum(C),
                                                       jnp.float32)
                    mref[0] = m_new
                    cb = [jnp.full((16,), C[r], jnp.float32)
                          for r in range(G)]

                    @plsc.parallel_loop(0, NCH)
                    def _upd(j):
                        p = [cb[r] * xbuf[gr + r, pl.ds(j * 16, 16)]
                             for r in range(4)]
                        for r in range(4, G):
                            p[r % 4] = (p[r % 4]
                                        + cb[r] * xbuf[gr + r,
                                                       pl.ds(j * 16, 16)])
                        old = cacc[pl.ds(j * 16, 16)] * f_v
                        cacc[pl.ds(j * 16, 16)] = (
                            ((p[0] + p[1]) + (p[2] + p[3])) + old)

                @pl.when(jnp.logical_not(no_boundary))
                def _():
                    def row(r, _):
                        g = g0 + r

                        @pl.when(g == nxt[0])
                        def _():
                            flush()

                        sc = jnp.max(jnp.where(iota16 == r, S, negs))
                        m_old = mref[0]
                        m_new = jnp.maximum(m_old, sc)
                        f_v = jnp.exp(jnp.full((16,), m_old - m_new,
                                               jnp.float32))
                        c_v = jnp.exp(jnp.full((16,), sc - m_new, jnp.float32))
                        sv[...] = sv[...] * f_v + c_v
                        mref[0] = m_new

                        def upd(j, _):
                            cacc[pl.ds(j * 16, 16)] = (
                                cacc[pl.ds(j * 16, 16)] * f_v
                                + c_v * xbuf[gr + r, pl.ds(j * 16, 16)])
                            return _
                        lax.fori_loop(0, NCH, upd, None)
                        return _
                    lax.fori_loop(0, G, row, None)

        pltpu.async_copy(x_hbm.at[pl.ds(t0, R), :], xbuf0, sem0)

        def tile_pair(kk, _):
            r1 = t0 + (2 * kk + 1) * R
            pltpu.async_copy(x_hbm.at[pl.ds(r1, R), :], xbuf1, sem1)
            pltpu.make_async_copy(x_hbm.at[pl.ds(0, R), :], xbuf0, sem0).wait()
            process(xbuf0, t0 + (2 * kk) * R)
            nx = t0 + jnp.minimum((2 * kk + 2) * R, (NTILES - 1) * R)
            pltpu.async_copy(x_hbm.at[pl.ds(nx, R), :], xbuf0, sem0)
            pltpu.make_async_copy(x_hbm.at[pl.ds(0, R), :], xbuf1, sem1).wait()
            process(xbuf1, t0 + (2 * kk + 1) * R)
            return _

        lax.fori_loop(0, NTILES // 2, tile_pair, None)
        pltpu.make_async_copy(x_hbm.at[pl.ds(0, R), :], xbuf0, sem0).wait()

        flush()

        pltpu.sync_copy(acc, acc_out.at[wid])
        pltpu.sync_copy(mloc, m_out.at[wid])
        pltpu.sync_copy(sloc, s_out.at[wid])

    return k(x, w, cu)


def _combine_kernel(acc_ref, m_ref, s_ref, out_ref):
    m = m_ref[...]
    s = s_ref[...]
    gmax = jnp.max(m, axis=0)
    f = jnp.exp(m - gmax[None, :])
    denom = jnp.sum(f * s, axis=0)
    f3 = f[:, :, None]
    wacc = jnp.sum(jnp.where(f3 > 0.0, acc_ref[...] * f3, 0.0),
                   axis=0)
    out_ref[...] = wacc / denom[:, None]


def kernel(x, cu_seqlens, W, b):
    w = W.reshape(-1)
    acc_p, m_p, s_p = _sc_partials(x, w, cu_seqlens.astype(jnp.int32))
    out = pl.pallas_call(
        _combine_kernel,
        out_shape=jax.ShapeDtypeStruct((NSEG, D), jnp.float32),
    )(acc_p, m_p, s_p)
    return out
